# baseline (speedup 1.0000x reference)
import jax
import jax.numpy as jnp
from jax import lax
from jax.experimental import pallas as pl
from jax.experimental.pallas import tpu as pltpu
from jax.experimental.pallas import tpu_sc as plsc

N = 10000
E = 320000
DF = 128
DE = 128
DH = 256
NPAD = 10112
TRASH = N

NC, NS, L = 2, 16, 16
NW = NC * NS

_SC_MESH = dict(core_axis_name="c", subcore_axis_name="s")


_EPW = E // NW
_EP_CHUNK = 2000


def _cat16(t):
  par = lax.rem(t, 2)
  return jnp.where(t < 5, 0,
                   jnp.where(t < 11, 2 - par, jnp.where(t < 15, 4 - par, 5)))


def _edge_prep_body(src_hbm, dst_hbm, ntypes_hbm,
                    maskf_hbm, idxm_hbm, idxnm_hbm,
                    src_v, dst_v, ts_v, td_v, mf_v, im_v, inm_v, sem):
  wid = lax.axis_index("s") * NC + lax.axis_index("c")
  base = wid * _EPW

  def chunk_body(ci, _):
    off = base + ci * _EP_CHUNK
    pltpu.sync_copy(src_hbm.at[pl.ds(off, _EP_CHUNK)], src_v)
    pltpu.sync_copy(dst_hbm.at[pl.ds(off, _EP_CHUNK)], dst_v)
    pltpu.async_copy(ntypes_hbm.at[src_v], ts_v, sem).wait()
    pltpu.async_copy(ntypes_hbm.at[dst_v], td_v, sem).wait()

    def vec_body(i, _):
      sl = pl.ds(i * L, L)
      m = _cat16(ts_v[sl]) == _cat16(td_v[sl])
      dv = dst_v[sl]
      mf_v[sl] = jnp.where(m, 1.0, 0.0).astype(jnp.float32)
      im_v[sl] = jnp.where(m, dv, TRASH).astype(jnp.int32)
      inm_v[sl] = jnp.where(m, TRASH, dv).astype(jnp.int32)
      return 0

    lax.fori_loop(0, _EP_CHUNK // L, vec_body, 0)
    pltpu.sync_copy(mf_v, maskf_hbm.at[pl.ds(off, _EP_CHUNK)])
    pltpu.sync_copy(im_v, idxm_hbm.at[pl.ds(off, _EP_CHUNK)])
    pltpu.sync_copy(inm_v, idxnm_hbm.at[pl.ds(off, _EP_CHUNK)])
    return 0

  lax.fori_loop(0, _EPW // _EP_CHUNK, chunk_body, 0)


def _edge_prep(src, dst, node_types):
  k = pl.kernel(
      _edge_prep_body,
      mesh=plsc.VectorSubcoreMesh(**_SC_MESH),
      out_type=(
          jax.ShapeDtypeStruct((E,), jnp.float32),
          jax.ShapeDtypeStruct((E,), jnp.int32),
          jax.ShapeDtypeStruct((E,), jnp.int32),
      ),
      scratch_types=[
          pltpu.VMEM((_EP_CHUNK,), jnp.int32),
          pltpu.VMEM((_EP_CHUNK,), jnp.int32),
          pltpu.VMEM((_EP_CHUNK,), jnp.int32),
          pltpu.VMEM((_EP_CHUNK,), jnp.int32),
          pltpu.VMEM((_EP_CHUNK,), jnp.float32),
          pltpu.VMEM((_EP_CHUNK,), jnp.int32),
          pltpu.VMEM((_EP_CHUNK,), jnp.int32),
          pltpu.SemaphoreType.DMA,
      ],
  )
  return k(src, dst, node_types)


_GPW = E // NW
_G_CHUNK = 200


def _gather_body(table_hbm, idxa_hbm, idxb_hbm, outa_hbm, outb_hbm,
                 idxa_v, idxb_v, rowsa_v, rowsb_v, sema, semb):
  wid = lax.axis_index("s") * NC + lax.axis_index("c")
  base = wid * _GPW

  def chunk_body(ci, _):
    off = base + ci * _G_CHUNK
    pltpu.sync_copy(idxa_hbm.at[pl.ds(off, _G_CHUNK)], idxa_v)
    ca = pltpu.async_copy(table_hbm.at[idxa_v], rowsa_v, sema)
    pltpu.sync_copy(idxb_hbm.at[pl.ds(off, _G_CHUNK)], idxb_v)
    cb = pltpu.async_copy(table_hbm.at[idxb_v], rowsb_v, semb)
    ca.wait()
    pltpu.sync_copy(rowsa_v, outa_hbm.at[pl.ds(off, _G_CHUNK)])
    cb.wait()
    pltpu.sync_copy(rowsb_v, outb_hbm.at[pl.ds(off, _G_CHUNK)])
    return 0

  lax.fori_loop(0, _GPW // _G_CHUNK, chunk_body, 0)


def _gather_rows(table, idxa, idxb):
  k = pl.kernel(
      _gather_body,
      mesh=plsc.VectorSubcoreMesh(**_SC_MESH),
      out_type=(
          jax.ShapeDtypeStruct((E, DF), jnp.float32),
          jax.ShapeDtypeStruct((E, DF), jnp.float32),
      ),
      scratch_types=[
          pltpu.VMEM((_G_CHUNK,), jnp.int32),
          pltpu.VMEM((_G_CHUNK,), jnp.int32),
          pltpu.VMEM((_G_CHUNK, DF), jnp.float32),
          pltpu.VMEM((_G_CHUNK, DF), jnp.float32),
          pltpu.SemaphoreType.DMA,
          pltpu.SemaphoreType.DMA,
      ],
  )
  return k(table, idxa, idxb)


_RPT = DF // NW
_S_CHUNK = 10000


def _scatter_body(valt_hbm, idx_hbm, out_hbm, idx_v, buf_v, acc_v, sem):
  wid = lax.axis_index("s") * NC + lax.axis_index("c")
  r0 = wid * _RPT
  z16 = jnp.zeros((L,), jnp.float32)

  def zloop(i, _):
    acc_v[pl.ds(i * L, L)] = z16
    return 0

  lax.fori_loop(0, _RPT * NPAD // L, zloop, 0)

  def chunk_body(ci, _):
    off = ci * _S_CHUNK
    pltpu.sync_copy(idx_hbm.at[pl.ds(off, _S_CHUNK)], idx_v)
    for r in range(_RPT):
      pltpu.sync_copy(valt_hbm.at[pl.ds((r0 + r) * E + off, _S_CHUNK)],
                      buf_v.at[pl.ds(r * _S_CHUNK, _S_CHUNK)])

    def vec_body(i, _):
      sl = pl.ds(i * L, L)
      dd = idx_v[sl]
      for r in range(_RPT):
        plsc.addupdate_scatter(acc_v, [dd + r * NPAD],
                               buf_v[pl.ds(r * _S_CHUNK + i * L, L)])
      return 0

    lax.fori_loop(0, _S_CHUNK // L, vec_body, 0)
    return 0

  lax.fori_loop(0, E // _S_CHUNK, chunk_body, 0)
  pltpu.sync_copy(acc_v, out_hbm.at[pl.ds(r0 * NPAD, _RPT * NPAD)])


def _segment_sum_t(valt, idx):
  k = pl.kernel(
      _scatter_body,
      mesh=plsc.VectorSubcoreMesh(**_SC_MESH),
      compiler_params=pltpu.CompilerParams(needs_layout_passes=False),
      out_type=jax.ShapeDtypeStruct((DF * NPAD,), jnp.float32),
      scratch_types=[
          pltpu.VMEM((_S_CHUNK,), jnp.int32),
          pltpu.VMEM((_RPT * _S_CHUNK,), jnp.float32),
          pltpu.VMEM((_RPT * NPAD,), jnp.float32),
          pltpu.SemaphoreType.DMA,
      ],
  )
  return k(valt.reshape(DF * E), idx).reshape(DF, NPAD)


_BN = 400
_BE = 512


def _dotg(a, b, dims, cast=0):
  if cast == 0:
    a = a.astype(jnp.bfloat16)
  else:
    b = b.astype(jnp.bfloat16)
  return jax.lax.dot_general(a, b, (dims, ((), ())),
                             preferred_element_type=jnp.float32)


def _node_enc_body(x_ref, w0_ref, b0_ref, w1_ref, b1_ref, o_ref):
  h = jnp.maximum(_dotg(x_ref[...], w0_ref[...], ((1,), (0,))) + b0_ref[...],
                  0.0)
  o_ref[...] = _dotg(h, w1_ref[...], ((1,), (0,))) + b1_ref[...]


def _node_enc(x, w0, b0, w1, b1):
  return pl.pallas_call(
      _node_enc_body,
      grid=(N // _BN,),
      in_specs=[
          pl.BlockSpec((_BN, DF), lambda i: (i, 0)),
          pl.BlockSpec((DF, DF), lambda i: (0, 0)),
          pl.BlockSpec((1, DF), lambda i: (0, 0)),
          pl.BlockSpec((DF, DF), lambda i: (0, 0)),
          pl.BlockSpec((1, DF), lambda i: (0, 0)),
      ],
      out_specs=pl.BlockSpec((_BN, DF), lambda i: (i, 0)),
      out_shape=jax.ShapeDtypeStruct((N, DF), jnp.float32),
  )(x, w0, b0[None, :], w1, b1[None, :])


def _edge_enc_body(ea_ref, w0_ref, b0_ref, w1_ref, b1_ref, o_ref):
  h = jnp.maximum(
      _dotg(w0_ref[...], ea_ref[...], ((0,), (1,)), cast=1) + b0_ref[...], 0.0)
  o_ref[...] = _dotg(w1_ref[...], h, ((0,), (0,)), cast=1) + b1_ref[...]


def _edge_enc(ea, w0, b0, w1, b1):
  return pl.pallas_call(
      _edge_enc_body,
      grid=(E // _BE,),
      in_specs=[
          pl.BlockSpec((_BE, 16), lambda i: (i, 0)),
          pl.BlockSpec((16, DE), lambda i: (0, 0)),
          pl.BlockSpec((DE, 1), lambda i: (0, 0)),
          pl.BlockSpec((DE, DE), lambda i: (0, 0)),
          pl.BlockSpec((DE, 1), lambda i: (0, 0)),
      ],
      out_specs=pl.BlockSpec((DE, _BE), lambda i: (0, i)),
      out_shape=jax.ShapeDtypeStruct((DE, E), jnp.float32),
  )(ea, w0, b0[:, None], w1, b1[:, None])


def _edge_mlp_sel_body(gs_ref, gd_ref, ea_ref, eb_ref, m_ref,
                       w1ab_ref, w1c_ref, b1_ref, w2_ref, b2_ref,
                       o_ref):
  m = m_ref[...]
  ef = ea_ref[...] * m + eb_ref[...] * (1.0 - m)
  gsd = jnp.concatenate([gs_ref[...], gd_ref[...]], axis=1)
  h = _dotg(w1ab_ref[...], gsd, ((0,), (1,)), cast=1)
  h += _dotg(w1c_ref[...], ef, ((0,), (0,)), cast=1)
  h = jnp.maximum(h + b1_ref[...], 0.0)
  o_ref[...] = jnp.maximum(_dotg(w2_ref[...], h, ((0,), (0,)), cast=1) + b2_ref[...],
                           0.0)


def _edge_mlp_body(gs_ref, gd_ref, ef_ref,
                   w1ab_ref, w1c_ref, b1_ref, w2_ref, b2_ref,
                   o_ref):
  gsd = jnp.concatenate([gs_ref[...], gd_ref[...]], axis=1)
  h = _dotg(w1ab_ref[...], gsd, ((0,), (1,)), cast=1)
  h += _dotg(w1c_ref[...], ef_ref[...], ((0,), (0,)), cast=1)
  h = jnp.maximum(h + b1_ref[...], 0.0)
  o_ref[...] = jnp.maximum(_dotg(w2_ref[...], h, ((0,), (0,)), cast=1) + b2_ref[...],
                           0.0)


def _edge_mlp(gs, gd, efa_t, efb_t, maskf2, p):
  w1ab = p['Wm1'][:2 * DF]
  w1c = p['Wm1'][2 * DF:]
  weight_specs = [
      pl.BlockSpec((2 * DF, DH), lambda i: (0, 0)),
      pl.BlockSpec((DE, DH), lambda i: (0, 0)),
      pl.BlockSpec((DH, 1), lambda i: (0, 0)),
      pl.BlockSpec((DH, DE), lambda i: (0, 0)),
      pl.BlockSpec((DE, 1), lambda i: (0, 0)),
  ]
  row_spec = pl.BlockSpec((_BE, DF), lambda i: (i, 0))
  et_spec = pl.BlockSpec((DE, _BE), lambda i: (0, i))
  args = (w1ab, w1c, p['bm1'][:, None], p['Wm2'], p['bm2'][:, None])
  if efb_t is None:
    return pl.pallas_call(
        _edge_mlp_body,
        grid=(E // _BE,),
        in_specs=[row_spec, row_spec, et_spec] + weight_specs,
        out_specs=pl.BlockSpec((DE, _BE), lambda i: (0, i)),
        out_shape=jax.ShapeDtypeStruct((DE, E), jnp.float32),
    )(gs, gd, efa_t, *args)
  return pl.pallas_call(
      _edge_mlp_sel_body,
      grid=(E // _BE,),
      in_specs=[row_spec, row_spec, et_spec, et_spec,
                pl.BlockSpec((1, _BE), lambda i: (0, i))] + weight_specs,
      out_specs=pl.BlockSpec((DE, _BE), lambda i: (0, i)),
      out_shape=jax.ShapeDtypeStruct((DE, E), jnp.float32),
  )(gs, gd, efa_t, efb_t, maskf2, *args)


def _node_update_body(nf_ref, aggt_ref, w_ref, b_ref, o_ref):
  agg = aggt_ref[...][:, :N].T
  cat = jnp.concatenate([nf_ref[...], agg], axis=1)
  h = _dotg(cat, w_ref[...], ((1,), (0,)))
  o_ref[...] = jnp.maximum(h + b_ref[...], 0.0)


def _node_update(nf, agg_t, p):
  return pl.pallas_call(
      _node_update_body,
      grid=(1,),
      in_specs=[
          pl.BlockSpec((N, DF), lambda i: (0, 0)),
          pl.BlockSpec((DE, NPAD), lambda i: (0, 0)),
          pl.BlockSpec((DF + DE, DF), lambda i: (0, 0)),
          pl.BlockSpec((1, DF), lambda i: (0, 0)),
      ],
      out_specs=pl.BlockSpec((N, DF), lambda i: (0, 0)),
      out_shape=jax.ShapeDtypeStruct((N, DF), jnp.float32),
  )(nf, agg_t, p['Wnu'], p['bnu'][None, :])


def _edge_head_body(ea_ref, eb_ref, m_ref, w0_ref, b0_ref, w1_ref, b1_ref,
                    o_ref):
  m = m_ref[...]
  ef = ea_ref[...] * m + eb_ref[...] * (1.0 - m)
  h = jnp.maximum(_dotg(w0_ref[...], ef, ((0,), (0,)), cast=1) + b0_ref[...], 0.0)
  o_ref[...] = _dotg(w1_ref[...], h, ((0,), (0,)), cast=1) + b1_ref[...]


def _edge_head(ea_t, eb_t, maskf2, p):
  out = pl.pallas_call(
      _edge_head_body,
      grid=(E // _BE,),
      in_specs=[
          pl.BlockSpec((DE, _BE), lambda i: (0, i)),
          pl.BlockSpec((DE, _BE), lambda i: (0, i)),
          pl.BlockSpec((1, _BE), lambda i: (0, i)),
          pl.BlockSpec((DE, 64), lambda i: (0, 0)),
          pl.BlockSpec((64, 1), lambda i: (0, 0)),
          pl.BlockSpec((64, 1), lambda i: (0, 0)),
          pl.BlockSpec((1, 1), lambda i: (0, 0)),
      ],
      out_specs=pl.BlockSpec((1, _BE), lambda i: (0, i)),
      out_shape=jax.ShapeDtypeStruct((1, E), jnp.float32),
  )(ea_t, eb_t, maskf2, p['Wec0'], p['bec0'][:, None], p['Wec1'],
    p['bec1'][:, None])
  return out[0]


def _node_heads_body(nf_ref, wn0_ref, bn0_ref, wn1_ref, bn1_ref,
                     wc0_ref, bc0_ref, wc1_ref, bc1_ref, on_ref, oc_ref):
  nf = nf_ref[...]
  h1 = jnp.maximum(_dotg(nf, wn0_ref[...], ((1,), (0,))) + bn0_ref[...], 0.0)
  on_ref[...] = _dotg(h1, wn1_ref[...], ((1,), (0,))) + bn1_ref[...]
  h2 = jnp.maximum(_dotg(nf, wc0_ref[...], ((1,), (0,))) + bc0_ref[...], 0.0)
  oc_ref[...] = _dotg(h2, wc1_ref[...], ((1,), (0,))) + bc1_ref[...]


def _node_heads(nf, p):
  return pl.pallas_call(
      _node_heads_body,
      grid=(N // _BN,),
      in_specs=[
          pl.BlockSpec((_BN, DF), lambda i: (i, 0)),
          pl.BlockSpec((DF, 64), lambda i: (0, 0)),
          pl.BlockSpec((1, 64), lambda i: (0, 0)),
          pl.BlockSpec((64, 1), lambda i: (0, 0)),
          pl.BlockSpec((1, 1), lambda i: (0, 0)),
          pl.BlockSpec((DF, 64), lambda i: (0, 0)),
          pl.BlockSpec((1, 64), lambda i: (0, 0)),
          pl.BlockSpec((64, 6), lambda i: (0, 0)),
          pl.BlockSpec((1, 6), lambda i: (0, 0)),
      ],
      out_specs=[
          pl.BlockSpec((_BN, 1), lambda i: (i, 0)),
          pl.BlockSpec((_BN, 6), lambda i: (i, 0)),
      ],
      out_shape=[
          jax.ShapeDtypeStruct((N, 1), jnp.float32),
          jax.ShapeDtypeStruct((N, 6), jnp.float32),
      ],
  )(nf, p['Wnc0'], p['bnc0'][None, :], p['Wnc1'], p['bnc1'][None, :],
    p['Wc0'], p['bc0'][None, :], p['Wc1'], p['bc1'][None, :])


def _layer(nf, efa_t, efb_t, maskf2, idx_scatter, src, dst, p):
  gs, gd = _gather_rows(nf, src, dst)
  new_e_t = _edge_mlp(gs, gd, efa_t, efb_t, maskf2, p)
  agg_t = _segment_sum_t(new_e_t, idx_scatter)
  nf_new = _node_update(nf, agg_t, p)
  return nf_new, new_e_t


def kernel(x, edge_attr, params, edge_index, node_types):
  p = params
  src = edge_index[0]
  dst = edge_index[1]
  maskf, idxm, idxnm = _edge_prep(src, dst, node_types)
  maskf2 = maskf[None, :]

  nf = _node_enc(x, p['Wn0'], p['bn0'], p['Wn1'], p['bn1'])
  ef_t = _edge_enc(edge_attr, p['We0'], p['be0'], p['We1'], p['be1'])

  nf, e1 = _layer(nf, ef_t, None, maskf2, idxm, src, dst, p)
  nf, e2 = _layer(nf, ef_t, None, maskf2, idxnm, src, dst, p)
  nf, e3 = _layer(nf, e1, e2, maskf2, idxm, src, dst, p)
  nf, e4 = _layer(nf, e1, e2, maskf2, idxnm, src, dst, p)

  pred_edge = _edge_head(e3, e4, maskf2, p)

  nf, ef_t = _layer(nf, e3, e4, maskf2, dst, src, dst, p)
  nf, ef_t = _layer(nf, ef_t, None, maskf2, dst, src, dst, p)

  pred_node, pred_cls = _node_heads(nf, p)
  return (pred_edge, pred_node[:, 0], pred_cls)
